# Initial kernel scaffold; baseline (speedup 1.0000x reference)
import jax
import jax.numpy as jnp
from jax.experimental import pallas as pl


def kernel(hidden_states, params):
    raise NotImplementedError("write your pallas kernel here")


# speedup vs baseline: 1.3433x; 1.3433x over previous
import functools

import jax
import jax.numpy as jnp
from jax import lax
from jax.experimental import pallas as pl
from jax.experimental.pallas import tpu as pltpu

_T = 128
_K = 3


def _ffn_kernel(te_ref, x_ref, w1_ref, b1_ref, w2_ref, b2_ref,
                confw_ref, phiw1_ref, phib1_ref, phiw2_ref, epb_ref,
                tw1_ref, tb1_ref, tw2_ref, tb2_ref,
                dw1_ref, db1_ref, dw2_ref, db2_ref, dw3_ref,
                rw1_ref, rb1_ref, rw2_ref, rb2_ref, rw3_ref, rb3_ref,
                wrow_ref, vrow_ref,
                eow_ref, stats_ref, pooled_ref):
    i = pl.program_id(0)
    e = te_ref[i]

    @pl.when(i == 0)
    def _():
        stats_ref[...] = jnp.zeros_like(stats_ref)
        pooled_ref[...] = jnp.zeros_like(pooled_ref)

    x = x_ref[...]
    h1 = jax.nn.gelu(
        jnp.dot(x, w1_ref[0], preferred_element_type=jnp.float32) + b1_ref[0])
    eo = jnp.dot(h1, w2_ref[0], preferred_element_type=jnp.float32) + b2_ref[0]

    v = vrow_ref[0, 0]
    wv = wrow_ref[0, 0]
    eow = eo * wv[:, None]
    eow_ref[...] = eow
    pooled_ref[...] += jnp.sum(eow, axis=0, keepdims=True)

    iota = lax.broadcasted_iota(jnp.int32, (1, 128), 1)
    conf = jax.nn.sigmoid(jnp.sum(eo * confw_ref[0], axis=1) + epb_ref[0, 0, 0])
    ph = jnp.maximum(
        jnp.dot(eo, phiw1_ref[0], preferred_element_type=jnp.float32)
        + phib1_ref[0], 0.0)
    phi = jax.nn.sigmoid(jnp.sum(ph * phiw2_ref[0], axis=1) + epb_ref[0, 0, 1])
    conf_s = jnp.sum(conf * v)
    phi_s = jnp.sum(phi * v)
    stats_ref[...] += (jnp.where(iota == 0, conf_s, 0.0)
                       + jnp.where(iota == 1, phi_s, 0.0))

    @pl.when(e == 1)
    def _triage():
        t1 = jnp.maximum(
            jnp.dot(eo, tw1_ref[...], preferred_element_type=jnp.float32)
            + tb1_ref[...], 0.0)
        tl = jnp.dot(t1, tw2_ref[...], preferred_element_type=jnp.float32) + tb2_ref[...]
        t = jax.nn.softmax(tl, axis=-1)
        stats_ref[...] += jnp.where(iota == 2, jnp.sum(t * v[:, None]), 0.0)

    @pl.when(e == 7)
    def _drug():
        d1 = jnp.maximum(
            jnp.dot(eo, dw1_ref[...], preferred_element_type=jnp.float32)
            + db1_ref[...], 0.0)
        d2 = jnp.maximum(
            jnp.dot(d1, dw2_ref[...], preferred_element_type=jnp.float32)
            + db2_ref[...], 0.0)
        d = jax.nn.sigmoid(jnp.sum(d2 * dw3_ref[...], axis=1) + epb_ref[0, 0, 2])
        stats_ref[...] += jnp.where(iota == 3, jnp.sum(d * v), 0.0)

    @pl.when(e == 11)
    def _risk():
        r1 = jnp.maximum(
            jnp.dot(eo, rw1_ref[...], preferred_element_type=jnp.float32)
            + rb1_ref[...], 0.0)
        r2 = jnp.maximum(
            jnp.dot(r1, rw2_ref[...], preferred_element_type=jnp.float32)
            + rb2_ref[...], 0.0)
        r = jax.nn.sigmoid(
            jnp.dot(r2, rw3_ref[...], preferred_element_type=jnp.float32)
            + rb3_ref[...])
        stats_ref[...] += jnp.where(iota == 4, jnp.sum(r * v[:, None]), 0.0)


def _run_ffn(tile_e, X, wrow, vrow, p, NT, PT):
    E, H, I = p["W1"].shape
    H2, H4 = H // 2, H // 4
    f32 = jnp.float32

    neg = jnp.float32(-1e30)
    tw2p = jnp.zeros((H2, 128), f32).at[:, :4].set(p["tW2"])
    tb2p = jnp.full((1, 128), neg).at[0, :4].set(p["tb2"])
    rw3p = jnp.zeros((H2, 128), f32).at[:, :10].set(p["rW3"])
    rb3p = jnp.full((1, 128), neg).at[0, :10].set(p["rb3"])
    epb = jnp.zeros((E, 1, 128), f32)
    epb = epb.at[:, 0, 0].set(p["confb"]).at[:, 0, 1].set(p["phib2"])
    epb = epb.at[:, 0, 2].set(jnp.broadcast_to(p["db3"], (E,)))

    grid_spec = pltpu.PrefetchScalarGridSpec(
        num_scalar_prefetch=1,
        grid=(NT,),
        in_specs=[
            pl.BlockSpec((_T, H), lambda i, te: (i, 0)),
            pl.BlockSpec((1, H, I), lambda i, te: (te[i], 0, 0)),
            pl.BlockSpec((1, 1, I), lambda i, te: (te[i], 0, 0)),
            pl.BlockSpec((1, I, H), lambda i, te: (te[i], 0, 0)),
            pl.BlockSpec((1, 1, H), lambda i, te: (te[i], 0, 0)),
            pl.BlockSpec((1, 1, H), lambda i, te: (te[i], 0, 0)),
            pl.BlockSpec((1, H, H4), lambda i, te: (te[i], 0, 0)),
            pl.BlockSpec((1, 1, H4), lambda i, te: (te[i], 0, 0)),
            pl.BlockSpec((1, 1, H4), lambda i, te: (te[i], 0, 0)),
            pl.BlockSpec((1, 1, 128), lambda i, te: (te[i], 0, 0)),
            pl.BlockSpec((H, H2), lambda i, te: (0, 0)),
            pl.BlockSpec((1, H2), lambda i, te: (0, 0)),
            pl.BlockSpec((H2, 128), lambda i, te: (0, 0)),
            pl.BlockSpec((1, 128), lambda i, te: (0, 0)),
            pl.BlockSpec((H, H2), lambda i, te: (0, 0)),
            pl.BlockSpec((1, H2), lambda i, te: (0, 0)),
            pl.BlockSpec((H2, H4), lambda i, te: (0, 0)),
            pl.BlockSpec((1, H4), lambda i, te: (0, 0)),
            pl.BlockSpec((1, H4), lambda i, te: (0, 0)),
            pl.BlockSpec((H, H), lambda i, te: (0, 0)),
            pl.BlockSpec((1, H), lambda i, te: (0, 0)),
            pl.BlockSpec((H, H2), lambda i, te: (0, 0)),
            pl.BlockSpec((1, H2), lambda i, te: (0, 0)),
            pl.BlockSpec((H2, 128), lambda i, te: (0, 0)),
            pl.BlockSpec((1, 128), lambda i, te: (0, 0)),
            pl.BlockSpec((1, 1, _T), lambda i, te: (i, 0, 0)),
            pl.BlockSpec((1, 1, _T), lambda i, te: (i, 0, 0)),
        ],
        out_specs=[
            pl.BlockSpec((_T, H), lambda i, te: (i, 0)),
            pl.BlockSpec((1, 128), lambda i, te: (0, 0)),
            pl.BlockSpec((1, H), lambda i, te: (0, 0)),
        ],
    )
    eow, stats, pooled = pl.pallas_call(
        _ffn_kernel,
        grid_spec=grid_spec,
        out_shape=[
            jax.ShapeDtypeStruct((PT, H), f32),
            jax.ShapeDtypeStruct((1, 128), f32),
            jax.ShapeDtypeStruct((1, H), f32),
        ],
        compiler_params=pltpu.CompilerParams(
            dimension_semantics=("arbitrary",),
            vmem_limit_bytes=100 * 1024 * 1024),
    )(tile_e, X, p["W1"], p["b1"][:, None, :], p["W2"], p["b2"][:, None, :],
      p["confW"][:, None, :], p["phiW1"], p["phib1"][:, None, :],
      p["phiW2"][:, None, :], epb,
      p["tW1"], p["tb1"].reshape(1, H2), tw2p, tb2p,
      p["dW1"], p["db1"].reshape(1, H2), p["dW2"], p["db2"].reshape(1, H4),
      p["dW3"].reshape(1, H4),
      p["rW1"], p["rb1"].reshape(1, H), p["rW2"], p["rb2"].reshape(1, H2),
      rw3p, rb3p,
      wrow, vrow)
    return eow, stats, pooled


def kernel(hidden_states, params):
    p = params
    b, s, h = hidden_states.shape
    n = b * s
    E = p["gW"].shape[1]
    f32 = jnp.float32
    tok = hidden_states.reshape(n, h)

    logits = tok @ p["gW"] + p["gb"]
    probs = jax.nn.softmax(logits, axis=-1)
    specialty_probs = probs.reshape(b, s, E)
    urgency = jax.nn.sigmoid(tok @ p["uW"] + p["ub"]).reshape(b, s)
    topv, topi = lax.top_k(probs, _K)
    ew = jax.nn.softmax(topv, axis=-1)

    e_flat = topi.reshape(-1).astype(jnp.int32)
    ew_flat = ew.reshape(-1)
    P = n * _K
    onehot = (e_flat[:, None] == jnp.arange(E, dtype=jnp.int32)[None, :])
    counts = jnp.sum(onehot, axis=0).astype(jnp.int32)
    wsum = jnp.sum(ew_flat[:, None] * onehot.astype(f32), axis=0)
    cnt_f = counts.astype(f32)
    w_e = jnp.where(counts > 0, wsum / jnp.maximum(cnt_f, 1.0), 0.0)

    offs = jnp.cumsum(counts) - counts
    pcounts = ((counts + _T - 1) // _T) * _T
    cum_p = jnp.cumsum(pcounts)
    poffs = cum_p - pcounts
    NT = P // _T + E
    PT = NT * _T
    rows = jnp.arange(PT, dtype=jnp.int32)
    row_e = jnp.clip(jnp.searchsorted(cum_p, rows, side="right"),
                     0, E - 1).astype(jnp.int32)
    local = rows - poffs[row_e]
    valid = local < counts[row_e]
    perm = jnp.argsort(e_flat, stable=True).astype(jnp.int32)
    src = offs[row_e] + jnp.minimum(local, jnp.maximum(counts[row_e] - 1, 0))
    pair_of_row = perm[src]
    token_of_row = jnp.where(valid, pair_of_row // _K, 0)
    wrow = jnp.where(valid, w_e[row_e], 0.0).reshape(NT, 1, _T)
    vrow = valid.astype(f32).reshape(NT, 1, _T)
    tile_e = row_e.reshape(NT, _T)[:, 0]

    X = tok[token_of_row]
    eow, stats, pooled_s = _run_ffn(tile_e, X, wrow, vrow, p, NT, PT)

    inv = jnp.zeros((P,), jnp.int32).at[perm].set(
        jnp.arange(P, dtype=jnp.int32))
    pos = poffs[e_flat] + (inv - offs[e_flat])
    out_rows = eow[pos].reshape(n, _K, h).sum(axis=1)

    conf_mean = stats[0, 0] / jnp.maximum(jnp.float32(P), 1.0)
    phi_prob_mean = stats[0, 1] / jnp.maximum(jnp.float32(P), 1.0)
    n1, n7, n11 = cnt_f[1], cnt_f[7], cnt_f[11]
    triage_mean = jnp.where(n1 > 0, stats[0, 2] / (jnp.maximum(n1, 1.0) * 4.0), 0.0)
    drug_mean = jnp.where(n7 > 0, stats[0, 3] / jnp.maximum(n7, 1.0), 0.0)
    risk_mean = jnp.where(n11 > 0, stats[0, 4] / (jnp.maximum(n11, 1.0) * 10.0), 0.0)

    pooled = pooled_s / jnp.float32(n)
    phi_score = jax.nn.sigmoid(
        jnp.maximum(pooled @ p["fW1"] + p["fb1"], 0.0) @ p["fW2"] + p["fb2"])
    scale = 1.0 - 0.8 * (phi_score > 0.7).astype(f32)
    output = out_rows.reshape(b, s, h) * scale[:, None, None]
    unc_in = pooled * scale[:, None]
    uncertainty = jax.nn.sigmoid(
        jnp.maximum(unc_in @ p["uncW1"] + p["uncb1"], 0.0) @ p["uncW2"]
        + p["uncb2"])

    return (output, specialty_probs, urgency, topi.reshape(b, s, _K),
            conf_mean, triage_mean, drug_mean, risk_mean, phi_prob_mean,
            phi_score, uncertainty)
